# Initial kernel scaffold; baseline (speedup 1.0000x reference)
import jax
import jax.numpy as jnp
from jax.experimental import pallas as pl


def kernel(graph, triples, node_embeddings, node_embeddings_bias, W1, b1, W2, b2, relations):
    raise NotImplementedError("write your pallas kernel here")


# speedup vs baseline: 12.6074x; 12.6074x over previous
import functools

import jax
import jax.numpy as jnp
from jax import lax
from jax.experimental import pallas as pl
from jax.experimental.pallas import tpu as pltpu
from jax.experimental.pallas import tpu_sc as plsc

N = 10000
NRELS = 18
RTOT = 2 * NRELS + 1
D = 128
NE = 320000
E2 = 2 * NE
NT = 30000

NC, NS = 2, 16
NW = NC * NS
EPW = E2 // NW
KB = 80
NBLK = EPW // KB
RPT = N // NS
ZROWS = 25
CHB = 50

NTP = 30720
KB2 = 120
NBLK2 = (NTP // NW) // KB2


def _transform1(emb, bias, W):
    def body(e_ref, b_ref, w_ref, o_ref):
        x = jnp.maximum(e_ref[...] + b_ref[...], 0.0)
        o_ref[0] = jnp.dot(x, w_ref[0], preferred_element_type=jnp.float32)
    return pl.pallas_call(
        body,
        grid=(RTOT,),
        in_specs=[
            pl.BlockSpec((N, D), lambda r: (0, 0)),
            pl.BlockSpec((1, D), lambda r: (0, 0)),
            pl.BlockSpec((1, D, D), lambda r: (r, 0, 0)),
        ],
        out_specs=pl.BlockSpec((1, N, D), lambda r: (r, 0, 0)),
        out_shape=jax.ShapeDtypeStruct((RTOT, N, D), jnp.float32),
    )(emb, bias, W)


def _transform(x, W):
    def body(x_ref, w_ref, o_ref):
        o_ref[0] = jnp.dot(x_ref[...], w_ref[0], preferred_element_type=jnp.float32)
    return pl.pallas_call(
        body,
        grid=(RTOT,),
        in_specs=[
            pl.BlockSpec((N, D), lambda r: (0, 0)),
            pl.BlockSpec((1, D, D), lambda r: (r, 0, 0)),
        ],
        out_specs=pl.BlockSpec((1, N, D), lambda r: (r, 0, 0)),
        out_shape=jax.ShapeDtypeStruct((RTOT, N, D), jnp.float32),
    )(x, W)


def _combine(acc, yself, degp, b):
    def body(a_ref, ys_ref, dg_ref, b_ref, o_ref):
        ones = jnp.ones((NW, 1), jnp.float32)
        degsum = lax.dot_general(dg_ref[...], ones, (((0,), (0,)), ((), ())),
                                 preferred_element_type=jnp.float32)
        inv = 1.0 / (1.0 + degsum)
        tot = (a_ref[0] + a_ref[1] + ys_ref[...]) * inv + b_ref[...]
        o_ref[...] = jnp.maximum(tot, 0.0)
    return pl.pallas_call(
        body,
        out_shape=jax.ShapeDtypeStruct((N, D), jnp.float32),
    )(acc, yself, degp, b.reshape(1, D))


def _score(hs, hr, ho, relations):
    CH = 3072
    def body(hs_ref, hr_ref, ho_ref, rel_ref, s_ref, p_ref):
        prod = hs_ref[...] * hr_ref[...] * ho_ref[...]
        ones = jnp.ones((D, 1), jnp.float32)
        s_ref[...] = lax.dot_general(prod, ones, (((1,), (0,)), ((), ())),
                                     preferred_element_type=jnp.float32)
        p_ref[...] = jnp.sum(rel_ref[...] * rel_ref[...]).reshape(1, 1)
    return pl.pallas_call(
        body,
        grid=(NTP // CH,),
        in_specs=[
            pl.BlockSpec((CH, D), lambda i: (i, 0)),
            pl.BlockSpec((CH, D), lambda i: (i, 0)),
            pl.BlockSpec((CH, D), lambda i: (i, 0)),
            pl.BlockSpec((NRELS, D), lambda i: (0, 0)),
        ],
        out_specs=[
            pl.BlockSpec((CH, 1), lambda i: (i, 0)),
            pl.BlockSpec((1, 1), lambda i: (0, 0)),
        ],
        out_shape=[
            jax.ShapeDtypeStruct((NTP, 1), jnp.float32),
            jax.ShapeDtypeStruct((1, 1), jnp.float32),
        ],
    )(hs, hr, ho, relations)


def _edge_agg(y_flat, eidx3, edst3, with_deg):
    mesh = plsc.VectorSubcoreMesh(core_axis_name="c", subcore_axis_name="s")
    out_type = [jax.ShapeDtypeStruct((NC, N, D), jnp.float32)]
    if with_deg:
        out_type.append(jax.ShapeDtypeStruct((NW, N), jnp.float32))
    scratch = [
        pltpu.VMEM_SHARED((N, D), jnp.float32),
        pltpu.VMEM((CHB, KB), jnp.int32),
        pltpu.VMEM((CHB, KB), jnp.int32),
        pltpu.VMEM((KB, D), jnp.float32),
        pltpu.VMEM((ZROWS, D), jnp.float32),
        pltpu.SemaphoreType.DMA,
    ]
    if with_deg:
        scratch.append(pltpu.VMEM((N,), jnp.float32))

    def body(y_hbm, ei_hbm, ed_hbm, *rest):
        if with_deg:
            acc_hbm, deg_hbm, acc_sh, idx_v, dst_v, rb0, zb, sem0, deg_v = rest
        else:
            acc_hbm, acc_sh, idx_v, dst_v, rb0, zb, sem0 = rest
        c = lax.axis_index("c")
        s = lax.axis_index("s")
        wid = s * NC + c
        zvec = jnp.zeros((16,), jnp.float32)

        def _zrow(i, cry):
            for j in range(D // 16):
                zb[i, pl.ds(j * 16, 16)] = zvec
            return cry
        lax.fori_loop(0, ZROWS, _zrow, 0)
        if with_deg:
            def _zdeg(i, cry):
                deg_v[pl.ds(i * 16, 16)] = zvec
                return cry
            lax.fori_loop(0, N // 16, _zdeg, 0)
        for k in range(RPT // ZROWS):
            pltpu.sync_copy(zb, acc_sh.at[pl.ds(s * RPT + k * ZROWS, ZROWS)])
        plsc.subcore_barrier()

        ones16 = jnp.ones((16,), jnp.float32)

        def chunk(cc, cry):
            pltpu.sync_copy(ei_hbm.at[wid].at[pl.ds(cc * CHB, CHB)], idx_v)
            pltpu.sync_copy(ed_hbm.at[wid].at[pl.ds(cc * CHB, CHB)], dst_v)

            def step(j, cry2):
                pltpu.async_copy(y_hbm.at[idx_v.at[j]], rb0, sem0).wait()
                pltpu.sync_copy(rb0, acc_sh.at[dst_v.at[j]], add=True)
                return cry2
            lax.fori_loop(0, CHB, step, 0)

            if with_deg:
                def dstep(j, cry2):
                    for kk in range(KB // 16):
                        idx16 = dst_v[j, pl.ds(kk * 16, 16)]
                        plsc.addupdate_scatter(deg_v, [idx16], ones16)
                    return cry2
                lax.fori_loop(0, CHB, dstep, 0)
            return cry
        lax.fori_loop(0, NBLK // CHB, chunk, 0)

        if with_deg:
            pltpu.sync_copy(deg_v, deg_hbm.at[wid])

        plsc.subcore_barrier()
        pltpu.sync_copy(acc_sh.at[pl.ds(s * RPT, RPT)],
                        acc_hbm.at[c].at[pl.ds(s * RPT, RPT)])

    fn = pl.kernel(body, out_type=tuple(out_type), mesh=mesh,
                   scratch_types=scratch,
                   compiler_params=pltpu.CompilerParams(use_tc_tiling_on_sc=False, needs_layout_passes=False))
    res = fn(y_flat, eidx3, edst3)
    return res if with_deg else res[0]


def _decoder_gather(x, relations, tsp, tpp, top):
    mesh = plsc.VectorSubcoreMesh(core_axis_name="c", subcore_axis_name="s")
    out_type = tuple(jax.ShapeDtypeStruct((NTP, D), jnp.float32) for _ in range(3))
    scratch = [
        pltpu.VMEM((NBLK2, KB2), jnp.int32),
        pltpu.VMEM((NBLK2, KB2), jnp.int32),
        pltpu.VMEM((NBLK2, KB2), jnp.int32),
        pltpu.VMEM((KB2, D), jnp.float32),
        pltpu.SemaphoreType.DMA,
    ]

    def body(x_hbm, rel_hbm, ts_hbm, tp_hbm, to_hbm,
             hs_hbm, hr_hbm, ho_hbm, ts_v, tp_v, to_v, rb, sem0):
        c = lax.axis_index("c")
        s = lax.axis_index("s")
        wid = s * NC + c
        base = wid * (NBLK2 * KB2)
        pltpu.sync_copy(ts_hbm.at[wid], ts_v)
        pltpu.sync_copy(tp_hbm.at[wid], tp_v)
        pltpu.sync_copy(to_hbm.at[wid], to_v)

        def step(j, cry):
            pltpu.async_copy(x_hbm.at[ts_v.at[j]], rb, sem0).wait()
            pltpu.sync_copy(rb, hs_hbm.at[pl.ds(base + j * KB2, KB2)])
            pltpu.async_copy(rel_hbm.at[tp_v.at[j]], rb, sem0).wait()
            pltpu.sync_copy(rb, hr_hbm.at[pl.ds(base + j * KB2, KB2)])
            pltpu.async_copy(x_hbm.at[to_v.at[j]], rb, sem0).wait()
            pltpu.sync_copy(rb, ho_hbm.at[pl.ds(base + j * KB2, KB2)])
            return cry
        lax.fori_loop(0, NBLK2, step, 0)

    fn = pl.kernel(body, out_type=out_type, mesh=mesh, scratch_types=scratch,
                   compiler_params=pltpu.CompilerParams(use_tc_tiling_on_sc=False, needs_layout_passes=False))
    return fn(x, relations, tsp, tpp, top)


def kernel(graph, triples, node_embeddings, node_embeddings_bias, W1, b1, W2, b2, relations):
    s_ = graph[:, 0]
    p_ = graph[:, 1]
    o_ = graph[:, 2]
    eidx3 = jnp.concatenate([p_ * N + s_, (p_ + NRELS) * N + o_]).reshape(NW, NBLK, KB)
    edst3 = jnp.concatenate([o_, s_]).reshape(NW, NBLK, KB)

    y1 = _transform1(node_embeddings, node_embeddings_bias, W1)
    acc1, degp = _edge_agg(y1.reshape(RTOT * N, D), eidx3, edst3, True)
    x1 = _combine(acc1, y1[2 * NRELS], degp, b1)

    y2 = _transform(x1, W2)
    acc2 = _edge_agg(y2.reshape(RTOT * N, D), eidx3, edst3, False)
    x2 = _combine(acc2, y2[2 * NRELS], degp, b2)

    tpad = jnp.zeros((NTP - NT,), jnp.int32)
    tsp = jnp.concatenate([triples[:, 0], tpad]).reshape(NW, NBLK2, KB2)
    tpp = jnp.concatenate([triples[:, 1], tpad]).reshape(NW, NBLK2, KB2)
    top = jnp.concatenate([triples[:, 2], tpad]).reshape(NW, NBLK2, KB2)
    hs, hr, ho = _decoder_gather(x2, relations, tsp, tpp, top)
    scores_pad, pen = _score(hs, hr, ho, relations)

    return scores_pad.reshape(NTP)[:NT], pen[0, 0], x2


# speedup vs baseline: 17.6834x; 1.4026x over previous
import functools

import jax
import jax.numpy as jnp
from jax import lax
from jax.experimental import pallas as pl
from jax.experimental.pallas import tpu as pltpu
from jax.experimental.pallas import tpu_sc as plsc

N = 10000
NRELS = 18
RTOT = 2 * NRELS + 1
D = 128
NE = 320000
E2 = 2 * NE
NT = 30000

NC, NS = 2, 16
NW = NC * NS
EPW = E2 // NW
KB = 80
NBLK = EPW // KB
RPT = N // NS
ZROWS = 25
CHB = 50

NTP = 30720
KB2 = 120
NBLK2 = (NTP // NW) // KB2


def _transform1(emb, bias, W):
    def body(e_ref, b_ref, w_ref, o_ref):
        x = jnp.maximum(e_ref[...] + b_ref[...], 0.0)
        o_ref[0] = jnp.dot(x, w_ref[0], preferred_element_type=jnp.float32)
    return pl.pallas_call(
        body,
        grid=(RTOT,),
        in_specs=[
            pl.BlockSpec((N, D), lambda r: (0, 0)),
            pl.BlockSpec((1, D), lambda r: (0, 0)),
            pl.BlockSpec((1, D, D), lambda r: (r, 0, 0)),
        ],
        out_specs=pl.BlockSpec((1, N, D), lambda r: (r, 0, 0)),
        out_shape=jax.ShapeDtypeStruct((RTOT, N, D), jnp.float32),
    )(emb, bias, W)


def _transform(x, W):
    def body(x_ref, w_ref, o_ref):
        o_ref[0] = jnp.dot(x_ref[...], w_ref[0], preferred_element_type=jnp.float32)
    return pl.pallas_call(
        body,
        grid=(RTOT,),
        in_specs=[
            pl.BlockSpec((N, D), lambda r: (0, 0)),
            pl.BlockSpec((1, D, D), lambda r: (r, 0, 0)),
        ],
        out_specs=pl.BlockSpec((1, N, D), lambda r: (r, 0, 0)),
        out_shape=jax.ShapeDtypeStruct((RTOT, N, D), jnp.float32),
    )(x, W)


def _combine(acc, yself, degp, b):
    def body(a_ref, ys_ref, dg_ref, b_ref, o_ref):
        ones = jnp.ones((NW, 1), jnp.float32)
        degsum = lax.dot_general(dg_ref[...], ones, (((0,), (0,)), ((), ())),
                                 preferred_element_type=jnp.float32)
        inv = 1.0 / (1.0 + degsum)
        tot = (a_ref[0] + a_ref[1] + ys_ref[...]) * inv + b_ref[...]
        o_ref[...] = jnp.maximum(tot, 0.0)
    return pl.pallas_call(
        body,
        out_shape=jax.ShapeDtypeStruct((N, D), jnp.float32),
    )(acc, yself, degp, b.reshape(1, D))


def _score(hs, hr, ho, relations):
    CH = 3072
    def body(hs_ref, hr_ref, ho_ref, rel_ref, s_ref, p_ref):
        prod = hs_ref[...] * hr_ref[...] * ho_ref[...]
        ones = jnp.ones((D, 1), jnp.float32)
        s_ref[...] = lax.dot_general(prod, ones, (((1,), (0,)), ((), ())),
                                     preferred_element_type=jnp.float32)
        p_ref[...] = jnp.sum(rel_ref[...] * rel_ref[...]).reshape(1, 1)
    return pl.pallas_call(
        body,
        grid=(NTP // CH,),
        in_specs=[
            pl.BlockSpec((CH, D), lambda i: (i, 0)),
            pl.BlockSpec((CH, D), lambda i: (i, 0)),
            pl.BlockSpec((CH, D), lambda i: (i, 0)),
            pl.BlockSpec((NRELS, D), lambda i: (0, 0)),
        ],
        out_specs=[
            pl.BlockSpec((CH, 1), lambda i: (i, 0)),
            pl.BlockSpec((1, 1), lambda i: (0, 0)),
        ],
        out_shape=[
            jax.ShapeDtypeStruct((NTP, 1), jnp.float32),
            jax.ShapeDtypeStruct((1, 1), jnp.float32),
        ],
    )(hs, hr, ho, relations)


def _edge_agg(y_flat, eidx3, edst3, with_deg):
    mesh = plsc.VectorSubcoreMesh(core_axis_name="c", subcore_axis_name="s")
    out_type = [jax.ShapeDtypeStruct((NC, N, D), jnp.float32)]
    if with_deg:
        out_type.append(jax.ShapeDtypeStruct((NW, N), jnp.float32))
    scratch = [
        pltpu.VMEM_SHARED((N, D), jnp.float32),
        pltpu.VMEM((CHB, KB), jnp.int32),
        pltpu.VMEM((CHB, KB), jnp.int32),
        pltpu.VMEM((KB, D), jnp.float32),
        pltpu.VMEM((KB, D), jnp.float32),
        pltpu.VMEM((ZROWS, D), jnp.float32),
        pltpu.SemaphoreType.DMA,
        pltpu.SemaphoreType.DMA,
    ]
    if with_deg:
        scratch.append(pltpu.VMEM((N,), jnp.float32))

    def body(y_hbm, ei_hbm, ed_hbm, *rest):
        if with_deg:
            acc_hbm, deg_hbm, acc_sh, idx_v, dst_v, rb0, rb1, zb, sem0, sem1, deg_v = rest
        else:
            acc_hbm, acc_sh, idx_v, dst_v, rb0, rb1, zb, sem0, sem1 = rest
        c = lax.axis_index("c")
        s = lax.axis_index("s")
        wid = s * NC + c
        zvec = jnp.zeros((16,), jnp.float32)

        def _zrow(i, cry):
            for j in range(D // 16):
                zb[i, pl.ds(j * 16, 16)] = zvec
            return cry
        lax.fori_loop(0, ZROWS, _zrow, 0)
        if with_deg:
            def _zdeg(i, cry):
                deg_v[pl.ds(i * 16, 16)] = zvec
                return cry
            lax.fori_loop(0, N // 16, _zdeg, 0)
        for k in range(RPT // ZROWS):
            pltpu.sync_copy(zb, acc_sh.at[pl.ds(s * RPT + k * ZROWS, ZROWS)])
        plsc.subcore_barrier()

        ones16 = jnp.ones((16,), jnp.float32)

        def chunk(cc, cry):
            pltpu.sync_copy(ei_hbm.at[wid].at[pl.ds(cc * CHB, CHB)], idx_v)
            pltpu.sync_copy(ed_hbm.at[wid].at[pl.ds(cc * CHB, CHB)], dst_v)

            pltpu.async_copy(y_hbm.at[idx_v.at[0]], rb0, sem0)

            def step2(t, cry2):
                b0 = 2 * t
                b1 = 2 * t + 1
                pltpu.async_copy(y_hbm.at[idx_v.at[b1]], rb1, sem1)
                pltpu.make_async_copy(y_hbm.at[idx_v.at[b0]], rb0, sem0).wait()
                pltpu.sync_copy(rb0, acc_sh.at[dst_v.at[b0]], add=True)
                if with_deg:
                    for kk in range(KB // 16):
                        idx16 = dst_v[b0, pl.ds(kk * 16, 16)]
                        plsc.addupdate_scatter(deg_v, [idx16], ones16)

                @pl.when(b1 + 1 < CHB)
                def _():
                    pltpu.async_copy(y_hbm.at[idx_v.at[b1 + 1]], rb0, sem0)
                pltpu.make_async_copy(y_hbm.at[idx_v.at[b1]], rb1, sem1).wait()
                pltpu.sync_copy(rb1, acc_sh.at[dst_v.at[b1]], add=True)
                if with_deg:
                    for kk in range(KB // 16):
                        idx16 = dst_v[b1, pl.ds(kk * 16, 16)]
                        plsc.addupdate_scatter(deg_v, [idx16], ones16)
                return cry2
            lax.fori_loop(0, CHB // 2, step2, 0)
            return cry
        lax.fori_loop(0, NBLK // CHB, chunk, 0)

        if with_deg:
            pltpu.sync_copy(deg_v, deg_hbm.at[wid])

        plsc.subcore_barrier()
        pltpu.sync_copy(acc_sh.at[pl.ds(s * RPT, RPT)],
                        acc_hbm.at[c].at[pl.ds(s * RPT, RPT)])

    fn = pl.kernel(body, out_type=tuple(out_type), mesh=mesh,
                   scratch_types=scratch,
                   compiler_params=pltpu.CompilerParams(use_tc_tiling_on_sc=False, needs_layout_passes=False))
    res = fn(y_flat, eidx3, edst3)
    return res if with_deg else res[0]


def _decoder_gather(x, relations, tsp, tpp, top):
    mesh = plsc.VectorSubcoreMesh(core_axis_name="c", subcore_axis_name="s")
    out_type = tuple(jax.ShapeDtypeStruct((NTP, D), jnp.float32) for _ in range(3))
    scratch = [
        pltpu.VMEM((NBLK2, KB2), jnp.int32),
        pltpu.VMEM((NBLK2, KB2), jnp.int32),
        pltpu.VMEM((NBLK2, KB2), jnp.int32),
        pltpu.VMEM((KB2, D), jnp.float32),
        pltpu.VMEM((KB2, D), jnp.float32),
        pltpu.VMEM((KB2, D), jnp.float32),
        pltpu.SemaphoreType.DMA,
        pltpu.SemaphoreType.DMA,
        pltpu.SemaphoreType.DMA,
    ]

    def body(x_hbm, rel_hbm, ts_hbm, tp_hbm, to_hbm,
             hs_hbm, hr_hbm, ho_hbm, ts_v, tp_v, to_v,
             rbs, rbr, rbo, sems, semr, semo):
        c = lax.axis_index("c")
        s = lax.axis_index("s")
        wid = s * NC + c
        base = wid * (NBLK2 * KB2)
        pltpu.sync_copy(ts_hbm.at[wid], ts_v)
        pltpu.sync_copy(tp_hbm.at[wid], tp_v)
        pltpu.sync_copy(to_hbm.at[wid], to_v)

        def step(j, cry):
            pltpu.async_copy(x_hbm.at[ts_v.at[j]], rbs, sems)
            pltpu.async_copy(rel_hbm.at[tp_v.at[j]], rbr, semr)
            pltpu.async_copy(x_hbm.at[to_v.at[j]], rbo, semo)
            pltpu.make_async_copy(x_hbm.at[ts_v.at[j]], rbs, sems).wait()
            pltpu.sync_copy(rbs, hs_hbm.at[pl.ds(base + j * KB2, KB2)])
            pltpu.make_async_copy(rel_hbm.at[tp_v.at[j]], rbr, semr).wait()
            pltpu.sync_copy(rbr, hr_hbm.at[pl.ds(base + j * KB2, KB2)])
            pltpu.make_async_copy(x_hbm.at[to_v.at[j]], rbo, semo).wait()
            pltpu.sync_copy(rbo, ho_hbm.at[pl.ds(base + j * KB2, KB2)])
            return cry
        lax.fori_loop(0, NBLK2, step, 0)

    fn = pl.kernel(body, out_type=out_type, mesh=mesh, scratch_types=scratch,
                   compiler_params=pltpu.CompilerParams(use_tc_tiling_on_sc=False, needs_layout_passes=False))
    return fn(x, relations, tsp, tpp, top)


def kernel(graph, triples, node_embeddings, node_embeddings_bias, W1, b1, W2, b2, relations):
    s_ = graph[:, 0]
    p_ = graph[:, 1]
    o_ = graph[:, 2]
    eidx3 = jnp.concatenate([p_ * N + s_, (p_ + NRELS) * N + o_]).reshape(NW, NBLK, KB)
    edst3 = jnp.concatenate([o_, s_]).reshape(NW, NBLK, KB)

    y1 = _transform1(node_embeddings, node_embeddings_bias, W1)
    acc1, degp = _edge_agg(y1.reshape(RTOT * N, D), eidx3, edst3, True)
    x1 = _combine(acc1, y1[2 * NRELS], degp, b1)

    y2 = _transform(x1, W2)
    acc2 = _edge_agg(y2.reshape(RTOT * N, D), eidx3, edst3, False)
    x2 = _combine(acc2, y2[2 * NRELS], degp, b2)

    tpad = jnp.zeros((NTP - NT,), jnp.int32)
    tsp = jnp.concatenate([triples[:, 0], tpad]).reshape(NW, NBLK2, KB2)
    tpp = jnp.concatenate([triples[:, 1], tpad]).reshape(NW, NBLK2, KB2)
    top = jnp.concatenate([triples[:, 2], tpad]).reshape(NW, NBLK2, KB2)
    hs, hr, ho = _decoder_gather(x2, relations, tsp, tpp, top)
    scores_pad, pen = _score(hs, hr, ho, relations)

    return scores_pad.reshape(NTP)[:NT], pen[0, 0], x2
